# Initial kernel scaffold; baseline (speedup 1.0000x reference)
import jax
import jax.numpy as jnp
from jax.experimental import pallas as pl


def kernel(adj_indices, adj_values, feat_rows, feat_cols, feat_values, weight, bias):
    raise NotImplementedError("write your pallas kernel here")


# speedup vs baseline: 3.6106x; 3.6106x over previous
import functools

import jax
import jax.numpy as jnp
from jax import lax
from jax.experimental import pallas as pl
from jax.experimental.pallas import tpu as pltpu
from jax.experimental.pallas import tpu_sc as plsc

N = 10000
C = 128
NC = 2
NS = 16
NW = NC * NS
L = 16
NP = 10240
ROWS_PER_TILE = NP // NS
K = 80

_mesh = plsc.VectorSubcoreMesh(core_axis_name="c", subcore_axis_name="s")


def _make_spmm(n_edges, n_table_rows):
    ne = n_edges // NW
    n_chunks = ne // K
    assert ne * NW == n_edges and n_chunks * K == ne

    @functools.partial(
        pl.kernel,
        mesh=_mesh,
        out_type=jax.ShapeDtypeStruct((NC, NP, C), jnp.float32),
        scratch_types=[
            pltpu.VMEM((K,), jnp.int32),
            pltpu.VMEM((K,), jnp.int32),
            pltpu.VMEM((K,), jnp.float32),
            pltpu.VMEM((K, C), jnp.float32),
            pltpu.VMEM((16, C), jnp.float32),
            pltpu.VMEM_SHARED((NP, C), jnp.float32),
            pltpu.SemaphoreType.DMA,
        ],
    )
    def spmm(table, dst, src, vals, out, dstv, srcv, valv, g, zbuf, acc, sem):
        cid = lax.axis_index("c")
        sid = lax.axis_index("s")
        wid = cid * NS + sid

        def _zrow(i, _):
            for j in range(C // L):
                zbuf[i, pl.ds(j * L, L)] = jnp.zeros((L,), jnp.float32)
            return 0
        lax.fori_loop(0, 16, _zrow, 0)

        def _zacc(r, _):
            pltpu.sync_copy(zbuf, acc.at[pl.ds(sid * ROWS_PER_TILE + r * 16, 16)])
            return 0
        lax.fori_loop(0, ROWS_PER_TILE // 16, _zacc, 0)
        plsc.subcore_barrier()

        base = wid * ne

        def _chunk(t, _):
            s0 = base + t * K
            pltpu.sync_copy(dst.at[pl.ds(s0, K)], dstv)
            pltpu.sync_copy(src.at[pl.ds(s0, K)], srcv)
            pltpu.sync_copy(vals.at[pl.ds(s0, K)], valv)
            pltpu.async_copy(table.at[srcv], g, sem).wait()

            def _scale(t2, _):
                v16 = valv[pl.ds(t2 * 16, 16)]
                for i in range(16):
                    vb = jnp.full((L,), v16[i], jnp.float32)
                    row = t2 * 16 + i
                    for j in range(C // L):
                        g[row, pl.ds(j * L, L)] = g[row, pl.ds(j * L, L)] * vb
                return 0
            lax.fori_loop(0, K // 16, _scale, 0)

            pltpu.sync_copy(g, acc.at[dstv], add=True)
            return 0
        lax.fori_loop(0, n_chunks, _chunk, 0)

        plsc.subcore_barrier()
        pltpu.sync_copy(
            acc.at[pl.ds(sid * ROWS_PER_TILE, ROWS_PER_TILE)],
            out.at[cid, pl.ds(sid * ROWS_PER_TILE, ROWS_PER_TILE)],
        )

    return spmm


_N_CHUNKS_OUT = N // 16
_CHUNKS_PER_W = -(-_N_CHUNKS_OUT // NW)


def _make_combine(with_bias_relu):
    def body(p, bias, out, a, b, biasv, sem):
        del sem
        cid = lax.axis_index("c")
        sid = lax.axis_index("s")
        wid = cid * NS + sid
        if with_bias_relu:
            pltpu.sync_copy(bias, biasv)

        def _chunk(t, _):
            ch = wid + t * NW

            @pl.when(ch < _N_CHUNKS_OUT)
            def _():
                r0 = ch * 16
                pltpu.sync_copy(p.at[0, pl.ds(r0, 16)], a)
                pltpu.sync_copy(p.at[1, pl.ds(r0, 16)], b)

                def _row(i, _):
                    for j in range(C // L):
                        x = a[i, pl.ds(j * L, L)] + b[i, pl.ds(j * L, L)]
                        if with_bias_relu:
                            x = jnp.maximum(x + biasv[0, pl.ds(j * L, L)], 0.0)
                        a[i, pl.ds(j * L, L)] = x
                    return 0
                lax.fori_loop(0, 16, _row, 0)
                pltpu.sync_copy(a, out.at[pl.ds(r0, 16)])
            return 0
        lax.fori_loop(0, _CHUNKS_PER_W, _chunk, 0)

    if with_bias_relu:
        fn = body
    else:
        def fn(p, out, a, b, biasv, sem):
            return body(p, None, out, a, b, biasv, sem)

    return pl.kernel(
        fn,
        mesh=_mesh,
        out_type=jax.ShapeDtypeStruct((N, C), jnp.float32),
        scratch_types=[
            pltpu.VMEM((16, C), jnp.float32),
            pltpu.VMEM((16, C), jnp.float32),
            pltpu.VMEM((1, C), jnp.float32),
            pltpu.SemaphoreType.DMA,
        ],
    )


_spmm_feat = _make_spmm(128000, C)
_spmm_adj = _make_spmm(320000, N)
_combine_relu = _make_combine(True)
_combine_plain = _make_combine(False)


def kernel(adj_indices, adj_values, feat_rows, feat_cols, feat_values, weight, bias):
    p = _spmm_feat(weight, feat_rows, feat_cols, feat_values)
    base = _combine_relu(p, bias)
    for _ in range(2):
        p = _spmm_adj(base, adj_indices[0], adj_indices[1], adj_values)
        base = _combine_plain(p)
    return base
